# baseline (speedup 1.0000x reference)
import functools

import jax
import jax.numpy as jnp
from jax import lax
from jax.experimental import pallas as pl
from jax.experimental.pallas import tpu as pltpu
from jax.experimental.pallas import tpu_sc as plsc

N = 10000
E = 160000
D = 256
G = 64
C = 10
RATIO = 0.8

NPAD = 10240
BLK = 1000
NBLK = N // BLK
ECH = 128
NWORK = 32
PER = 40
EROWS = NWORK * PER
EPAD = EROWS * ECH
NACC = N + 16
RPS = 632
RPSL = N - 15 * RPS
RPSZ = NACC - 15 * RPS


NSLOT = 2
P0 = 48
P1 = 80 - P0
PMAX = max(P0, P1)


def _make_sc_body(count_mode):

    def body(*refs):
        it = iter(refs)
        table = next(it)
        srcm = next(it)
        dstm = next(it)
        zrows = next(it)
        zc = cntsrc = onesrc = None
        if count_mode != "none":
            zc = next(it)
        if count_mode == "ones":
            onesrc = next(it)
        if count_mode == "gather":
            cntsrc = next(it)
        out = next(it)
        outc = next(it) if count_mode != "none" else None
        sidx = [next(it) for _ in range(NSLOT)]
        didx = [next(it) for _ in range(NSLOT)]
        bufs = [next(it) for _ in range(NSLOT)]
        gsem = [next(it) for _ in range(NSLOT)]
        isem = [next(it) for _ in range(NSLOT)]
        acc = next(it)
        accc = None
        cbufs = []
        if count_mode != "none":
            accc = next(it)
            nb = 1 if count_mode == "ones" else NSLOT
            cbufs = [next(it) for _ in range(nb)]

        c = lax.axis_index("c")
        s = lax.axis_index("s")
        nj = jnp.where(c == 0, P0, P1)
        r0 = jnp.where(c == 0, s * P0, 16 * P0 + s * P1)
        z0 = s * RPS

        @pl.when(s < 15)
        def _():
            pltpu.sync_copy(zrows.at[pl.ds(0, RPS)], acc.at[pl.ds(z0, RPS)])
            if count_mode != "none":
                pltpu.sync_copy(zc.at[pl.ds(0, RPS)], accc.at[pl.ds(z0, RPS)])

        @pl.when(s == 15)
        def _():
            pltpu.sync_copy(zrows.at[pl.ds(0, RPSZ)], acc.at[pl.ds(15 * RPS, RPSZ)])
            if count_mode != "none":
                pltpu.sync_copy(zc.at[pl.ds(0, RPSZ)], accc.at[pl.ds(15 * RPS, RPSZ)])

        if count_mode == "ones":
            pltpu.sync_copy(onesrc, cbufs[0])

        def issue_idx(j, b):
            pltpu.async_copy(srcm.at[r0 + j], sidx[b], isem[b])
            pltpu.async_copy(dstm.at[r0 + j], didx[b], isem[b])

        def wait_idx(b):
            pltpu.make_async_copy(srcm.at[r0], sidx[b], isem[b]).wait()
            pltpu.make_async_copy(dstm.at[r0], didx[b], isem[b]).wait()

        def issue_gather(b):
            pltpu.async_copy(table.at[sidx[b]], bufs[b], gsem[b])
            if count_mode == "gather":
                pltpu.async_copy(cntsrc.at[sidx[b]], cbufs[b], gsem[b])

        def wait_gather(b):
            pltpu.make_async_copy(table.at[sidx[0]], bufs[b], gsem[b]).wait()
            if count_mode == "gather":
                pltpu.make_async_copy(cntsrc.at[sidx[0]], cbufs[b], gsem[b]).wait()

        def scatter(j, b):
            pltpu.sync_copy(bufs[b], acc.at[didx[b]], add=True)
            if count_mode == "ones":
                pltpu.sync_copy(cbufs[0], accc.at[didx[b]], add=True)
            elif count_mode == "gather":
                pltpu.sync_copy(cbufs[b], accc.at[didx[b]], add=True)

        plsc.subcore_barrier()

        issue_idx(0, 0)
        issue_idx(1, 1)
        wait_idx(0)
        issue_gather(0)

        def step(j2, carry):
            for b in range(NSLOT):
                j = j2 * NSLOT + b
                nb = 1 - b

                @pl.when(j < nj)
                def _():
                    @pl.when(j + 1 < nj)
                    def _():
                        wait_idx(nb)
                        issue_gather(nb)

                    wait_gather(b)
                    scatter(j, b)

                    @pl.when(j + NSLOT < nj)
                    def _():
                        issue_idx(j + NSLOT, b)
            return carry

        lax.fori_loop(0, (PMAX + NSLOT - 1) // NSLOT, step, 0)

        plsc.subcore_barrier()

        @pl.when(s < 15)
        def _():
            pltpu.sync_copy(acc.at[pl.ds(z0, RPS)], out.at[c, pl.ds(z0, RPS)])
            if count_mode != "none":
                pltpu.sync_copy(accc.at[pl.ds(z0, RPS)], outc.at[c, pl.ds(z0, RPS)])

        @pl.when(s == 15)
        def _():
            pltpu.sync_copy(
                acc.at[pl.ds(15 * RPS, RPSL)], out.at[c, pl.ds(15 * RPS, RPSL)]
            )
            if count_mode != "none":
                pltpu.sync_copy(
                    accc.at[pl.ds(15 * RPS, RPSL)], outc.at[c, pl.ds(15 * RPS, RPSL)]
                )

    return body


@functools.cache
def _sc_agg(count_mode):
    mesh = plsc.VectorSubcoreMesh(
        core_axis_name="c", subcore_axis_name="s", num_cores=2, num_subcores=16
    )
    ncb = 0 if count_mode == "none" else (1 if count_mode == "ones" else NSLOT)
    out_type = [jax.ShapeDtypeStruct((2, N, 128), jnp.float32)]
    if count_mode != "none":
        out_type.append(jax.ShapeDtypeStruct((2, N, 16), jnp.float32))
    scratch = [pltpu.VMEM((ECH,), jnp.int32) for _ in range(2 * NSLOT)]
    scratch += [pltpu.VMEM((ECH, 128), jnp.float32) for _ in range(NSLOT)]
    scratch += [pltpu.SemaphoreType.DMA for _ in range(2 * NSLOT)]
    scratch += [pltpu.VMEM_SHARED((NACC, 128), jnp.float32)]
    if count_mode != "none":
        scratch += [pltpu.VMEM_SHARED((NACC, 16), jnp.float32)]
        scratch += [pltpu.VMEM((ECH, 16), jnp.float32) for _ in range(ncb)]
    return pl.kernel(
        _make_sc_body(count_mode),
        out_type=out_type,
        mesh=mesh,
        scratch_types=scratch,
        compiler_params=pltpu.CompilerParams(use_tc_tiling_on_sc=False),
    )


def _k1_body(x_ref, wl_ref, wr_ref, p_ref, r_ref):
    xb = x_ref[...]
    p_ref[...] = jnp.dot(xb, wl_ref[...], preferred_element_type=jnp.float32)
    r_ref[...] = jnp.dot(xb, wr_ref[...], preferred_element_type=jnp.float32)


@functools.cache
def _k1():
    return pl.pallas_call(
        _k1_body,
        grid=(NBLK,),
        in_specs=[
            pl.BlockSpec((BLK, D), lambda i: (i, 0)),
            pl.BlockSpec((D, 128), lambda i: (0, 0)),
            pl.BlockSpec((D, 128), lambda i: (0, 0)),
        ],
        out_specs=[
            pl.BlockSpec((BLK, 128), lambda i: (i, 0)),
            pl.BlockSpec((BLK, 128), lambda i: (i, 0)),
        ],
        out_shape=[
            jax.ShapeDtypeStruct((N, 128), jnp.float32),
            jax.ShapeDtypeStruct((N, 128), jnp.float32),
        ],
    )


def _k2_body(parts_ref, cp_ref, r1_ref, wl_ref, wr_ref, b1_ref,
             p2_ref, r2_ref, cnt_ref):
    ps = parts_ref[0] + parts_ref[1]
    cnt = cp_ref[0, :, :1] + cp_ref[1, :, :1]
    agg = ps / jnp.maximum(cnt, 1.0)
    h = jax.nn.relu(agg + b1_ref[...] + r1_ref[...])
    p2_ref[...] = jnp.dot(h, wl_ref[...], preferred_element_type=jnp.float32)
    r2_ref[...] = jnp.dot(h, wr_ref[...], preferred_element_type=jnp.float32)
    cnt_ref[...] = cnt


@functools.cache
def _k2():
    return pl.pallas_call(
        _k2_body,
        grid=(NBLK,),
        in_specs=[
            pl.BlockSpec((2, BLK, 128), lambda i: (0, i, 0)),
            pl.BlockSpec((2, BLK, 16), lambda i: (0, i, 0)),
            pl.BlockSpec((BLK, 128), lambda i: (i, 0)),
            pl.BlockSpec((128, 128), lambda i: (0, 0)),
            pl.BlockSpec((128, 128), lambda i: (0, 0)),
            pl.BlockSpec((1, 128), lambda i: (0, 0)),
        ],
        out_specs=[
            pl.BlockSpec((BLK, 128), lambda i: (i, 0)),
            pl.BlockSpec((BLK, 128), lambda i: (i, 0)),
            pl.BlockSpec((BLK, 1), lambda i: (i, 0)),
        ],
        out_shape=[
            jax.ShapeDtypeStruct((N, 128), jnp.float32),
            jax.ShapeDtypeStruct((N, 128), jnp.float32),
            jax.ShapeDtypeStruct((N, 1), jnp.float32),
        ],
    )


def _k3a_body(parts_ref, r2_ref, cnt_ref, b2_ref, p_ref, batch_ref,
              h2_ref, s_ref, counts_ref):
    i = pl.program_id(0)
    ps = parts_ref[0] + parts_ref[1]
    agg = ps / jnp.maximum(cnt_ref[...], 1.0)
    h2 = jax.nn.relu(agg + b2_ref[...] + r2_ref[...])
    h2_ref[...] = h2
    p = p_ref[...]
    norm = jnp.sqrt(jnp.sum(p * p))
    pd = jnp.sum(h2 * p, axis=1, keepdims=True)
    s_ref[...] = jnp.tanh(pd / (norm + 1e-16))
    bt = batch_ref[0]
    gi = lax.broadcasted_iota(jnp.int32, (G, 1), 0)
    blkcnt = jnp.sum((bt == gi).astype(jnp.float32), axis=1, keepdims=True)

    @pl.when(i == 0)
    def _():
        counts_ref[...] = jnp.zeros_like(counts_ref)

    counts_ref[...] += blkcnt


@functools.cache
def _k3a():
    return pl.pallas_call(
        _k3a_body,
        grid=(NBLK,),
        in_specs=[
            pl.BlockSpec((2, BLK, 128), lambda i: (0, i, 0)),
            pl.BlockSpec((BLK, 128), lambda i: (i, 0)),
            pl.BlockSpec((BLK, 1), lambda i: (i, 0)),
            pl.BlockSpec((1, 128), lambda i: (0, 0)),
            pl.BlockSpec((1, 128), lambda i: (0, 0)),
            pl.BlockSpec((1, 1, BLK), lambda i: (i, 0, 0)),
        ],
        out_specs=[
            pl.BlockSpec((BLK, 128), lambda i: (i, 0)),
            pl.BlockSpec((BLK, 1), lambda i: (i, 0)),
            pl.BlockSpec((G, 1), lambda i: (0, 0)),
        ],
        out_shape=[
            jax.ShapeDtypeStruct((N, 128), jnp.float32),
            jax.ShapeDtypeStruct((N, 1), jnp.float32),
            jax.ShapeDtypeStruct((G, 1), jnp.float32),
        ],
    )


RBLK = 256
CCH = 512


def _krank_body(sc_ref, bc_ref, sr_ref, br_ref, counts_ref, keep_ref):
    i = pl.program_id(0)
    sc = sc_ref[...]
    bc = bc_ref[...]
    ivec = lax.broadcasted_iota(jnp.int32, (RBLK, 1), 0) + i * RBLK
    counts = counts_ref[...]
    ga = lax.broadcasted_iota(jnp.int32, (G, G), 0)
    gb = lax.broadcasted_iota(jnp.int32, (G, G), 1)
    ltri = (gb <= ga).astype(jnp.float32)
    cum = jnp.dot(ltri, counts, preferred_element_type=jnp.float32)
    startv = cum - counts
    gi = lax.broadcasted_iota(jnp.int32, (G, 1), 0)
    bmin = jnp.min(jnp.where(bc >= 0, bc, G - 1))
    bmax = jnp.max(bc)
    s_lo = jnp.sum(jnp.where(gi == bmin, startv, 0.0))
    e_hi = jnp.sum(jnp.where(gi == bmax, cum, 0.0))
    c_lo = (s_lo.astype(jnp.int32)) // CCH
    c_hi = (e_hi.astype(jnp.int32) + CCH - 1) // CCH

    def col_chunk(cc, acc):
        off = pl.multiple_of(cc * CCH, CCH)
        sj = sr_ref[:, pl.ds(off, CCH)]
        bj = br_ref[:, pl.ds(off, CCH)]
        jvec = lax.broadcasted_iota(jnp.int32, (1, CCH), 1) + cc * CCH
        beats = jnp.logical_and(
            bj == bc,
            jnp.logical_or(sj > sc, jnp.logical_and(sj == sc, jvec < ivec)),
        )
        return acc + jnp.sum(beats.astype(jnp.float32), axis=1, keepdims=True)

    acc = lax.fori_loop(c_lo, c_hi, col_chunk, jnp.zeros((RBLK, 1), jnp.float32))
    gj = lax.broadcasted_iota(jnp.int32, (1, G), 1)
    onehot = (bc == gj).astype(jnp.float32)
    kg = jnp.ceil(RATIO * counts)
    kat = jnp.dot(onehot, kg, preferred_element_type=jnp.float32)
    keep_ref[...] = (acc < kat).astype(jnp.float32)


@functools.cache
def _krank():
    return pl.pallas_call(
        _krank_body,
        grid=(NPAD // RBLK,),
        in_specs=[
            pl.BlockSpec((RBLK, 1), lambda i: (i, 0)),
            pl.BlockSpec((RBLK, 1), lambda i: (i, 0)),
            pl.BlockSpec((1, NPAD), lambda i: (0, 0)),
            pl.BlockSpec((1, NPAD), lambda i: (0, 0)),
            pl.BlockSpec((G, 1), lambda i: (0, 0)),
        ],
        out_specs=pl.BlockSpec((RBLK, 1), lambda i: (i, 0)),
        out_shape=jax.ShapeDtypeStruct((NPAD, 1), jnp.float32),
    )


def _k3c_body(h2_ref, s_ref, keep_ref, wl_ref, wr_ref, p3_ref, r_ref, k16_ref):
    keep = keep_ref[...]
    xp = h2_ref[...] * s_ref[...] * keep
    p3_ref[...] = jnp.dot(xp, wl_ref[...], preferred_element_type=jnp.float32)
    r_ref[...] = jnp.dot(xp, wr_ref[...], preferred_element_type=jnp.float32)
    k16_ref[...] = jnp.broadcast_to(keep, (BLK, 16))


@functools.cache
def _k3c():
    return pl.pallas_call(
        _k3c_body,
        grid=(NBLK,),
        in_specs=[
            pl.BlockSpec((BLK, 128), lambda i: (i, 0)),
            pl.BlockSpec((BLK, 1), lambda i: (i, 0)),
            pl.BlockSpec((BLK, 1), lambda i: (i, 0)),
            pl.BlockSpec((128, 128), lambda i: (0, 0)),
            pl.BlockSpec((128, 128), lambda i: (0, 0)),
        ],
        out_specs=[
            pl.BlockSpec((BLK, 128), lambda i: (i, 0)),
            pl.BlockSpec((BLK, 128), lambda i: (i, 0)),
            pl.BlockSpec((BLK, 16), lambda i: (i, 0)),
        ],
        out_shape=[
            jax.ShapeDtypeStruct((N, 128), jnp.float32),
            jax.ShapeDtypeStruct((N, 128), jnp.float32),
            jax.ShapeDtypeStruct((N, 16), jnp.float32),
        ],
    )


def _k4_body(parts_ref, cp_ref, r3_ref, keep_ref, batch_ref, b3_ref,
             wl1_ref, bl1_ref, wl2_ref, bl2_ref, out_ref, sums_ref, cnt2_ref):
    i = pl.program_id(0)
    ps = parts_ref[0] + parts_ref[1]
    cnt3 = cp_ref[0, :, :1] + cp_ref[1, :, :1]
    agg = ps / jnp.maximum(cnt3, 1.0)
    keep = keep_ref[...]
    h3 = jax.nn.relu(agg + b3_ref[...] + r3_ref[...]) * keep
    bt = batch_ref[0]
    gi = lax.broadcasted_iota(jnp.int32, (G, 1), 0)
    onehot = (bt == gi).astype(jnp.float32)
    bsums = jnp.dot(onehot, h3, preferred_element_type=jnp.float32)
    bcnt = jnp.dot(onehot, keep, preferred_element_type=jnp.float32)

    @pl.when(i == 0)
    def _():
        sums_ref[...] = jnp.zeros_like(sums_ref)
        cnt2_ref[...] = jnp.zeros_like(cnt2_ref)

    sums_ref[...] += bsums
    cnt2_ref[...] += bcnt

    @pl.when(i == NBLK - 1)
    def _():
        gm = sums_ref[...] / jnp.maximum(cnt2_ref[...], 1.0)
        o = jax.nn.relu(
            jnp.dot(gm, wl1_ref[...], preferred_element_type=jnp.float32)
            + bl1_ref[...]
        )
        z = jnp.dot(o, wl2_ref[...], preferred_element_type=jnp.float32) + bl2_ref[...]
        m = jnp.max(z, axis=1, keepdims=True)
        ez = jnp.exp(z - m)
        out_ref[...] = z - m - jnp.log(jnp.sum(ez, axis=1, keepdims=True))


@functools.cache
def _k4():
    return pl.pallas_call(
        _k4_body,
        grid=(NBLK,),
        in_specs=[
            pl.BlockSpec((2, BLK, 128), lambda i: (0, i, 0)),
            pl.BlockSpec((2, BLK, 16), lambda i: (0, i, 0)),
            pl.BlockSpec((BLK, 128), lambda i: (i, 0)),
            pl.BlockSpec((BLK, 1), lambda i: (i, 0)),
            pl.BlockSpec((1, 1, BLK), lambda i: (i, 0, 0)),
            pl.BlockSpec((1, 128), lambda i: (0, 0)),
            pl.BlockSpec((128, 64), lambda i: (0, 0)),
            pl.BlockSpec((1, 64), lambda i: (0, 0)),
            pl.BlockSpec((64, C), lambda i: (0, 0)),
            pl.BlockSpec((1, C), lambda i: (0, 0)),
        ],
        out_specs=pl.BlockSpec((G, C), lambda i: (0, 0)),
        out_shape=jax.ShapeDtypeStruct((G, C), jnp.float32),
        scratch_shapes=[
            pltpu.VMEM((G, 128), jnp.float32),
            pltpu.VMEM((G, 1), jnp.float32),
        ],
    )


def _agg1(table, srcm, dstm):
    zrows = jnp.zeros((RPS, 128), jnp.float32)
    zc = jnp.zeros((RPS, 16), jnp.float32)
    onesrc = jnp.ones((ECH, 16), jnp.float32)
    return _sc_agg("ones")(table, srcm, dstm, zrows, zc, onesrc)


def _agg2(table, srcm, dstm):
    zrows = jnp.zeros((RPS, 128), jnp.float32)
    return _sc_agg("none")(table, srcm, dstm, zrows)[0]


def _agg3(table, keep16, srcm, dstm):
    zrows = jnp.zeros((RPS, 128), jnp.float32)
    zc = jnp.zeros((RPS, 16), jnp.float32)
    return _sc_agg("gather")(table, srcm, dstm, zrows, zc, keep16)


def _pad_edges(ei):
    src = jnp.pad(ei[0], (0, EPAD - E)).reshape(EROWS, ECH)
    dst = jnp.pad(ei[1], (0, EPAD - E), constant_values=N).reshape(EROWS, ECH)
    return src, dst


def kernel(x, edge_index, batch, W1l, b1, W1r, W2l, b2, W2r, p_pool,
           W3l, b3, W3r, Wl1, bl1, Wl2, bl2):
    srcm, dstm = _pad_edges(edge_index)
    batch_row = batch.reshape(1, N)
    batch3 = batch.reshape(NBLK, 1, BLK)

    p1, r1 = _k1()(x, W1l, W1r)
    parts1, cnt1p = _agg1(p1, srcm, dstm)
    p2, r2, cnt1 = _k2()(parts1, cnt1p, r1, W2l, W2r, b1.reshape(1, 128))
    parts2 = _agg2(p2, srcm, dstm)
    h2, s_col, counts = _k3a()(
        parts2, r2, cnt1, b2.reshape(1, 128), p_pool.reshape(1, 128), batch3
    )

    s_row_pad = jnp.pad(s_col.reshape(1, N), ((0, 0), (0, NPAD - N)))
    s_col_pad = jnp.pad(s_col, ((0, NPAD - N), (0, 0)))
    batch_row_pad = jnp.pad(batch_row, ((0, 0), (0, NPAD - N)), constant_values=-1)
    batch_col_pad = batch_row_pad.reshape(NPAD, 1)
    keep_pad = _krank()(s_col_pad, batch_col_pad, s_row_pad, batch_row_pad, counts)
    keep = keep_pad[:N]

    p3, r3, keep16 = _k3c()(h2, s_col, keep, W3l, W3r)
    parts3, cnt3p = _agg3(p3, keep16, srcm, dstm)
    out = _k4()(
        parts3, cnt3p, r3, keep, batch3, b3.reshape(1, 128),
        Wl1, bl1.reshape(1, 64), Wl2, bl2.reshape(1, C),
    )
    return out


# speedup vs baseline: 1.0183x; 1.0183x over previous
import functools

import jax
import jax.numpy as jnp
from jax import lax
from jax.experimental import pallas as pl
from jax.experimental.pallas import tpu as pltpu
from jax.experimental.pallas import tpu_sc as plsc

N = 10000
E = 160000
D = 256
G = 64
C = 10
RATIO = 0.8

NPAD = 10240
BLK = 1000
NBLK = N // BLK
ECH = 128
NWORK = 32
PER = 40
EROWS = NWORK * PER
EPAD = EROWS * ECH
NACC = N + 16
RPS = 632
RPSL = N - 15 * RPS
RPSZ = NACC - 15 * RPS


NSLOT = 2
P0 = 59
P1 = 80 - P0
PMAX = max(P0, P1)


def _make_sc_body(count_mode):

    def body(*refs):
        it = iter(refs)
        table = next(it)
        srcm = next(it)
        dstm = next(it)
        zrows = next(it)
        zc = cntsrc = onesrc = None
        if count_mode != "none":
            zc = next(it)
        if count_mode == "ones":
            onesrc = next(it)
        if count_mode == "gather":
            cntsrc = next(it)
        out = next(it)
        outc = next(it) if count_mode != "none" else None
        sidx = [next(it) for _ in range(NSLOT)]
        didx = [next(it) for _ in range(NSLOT)]
        bufs = [next(it) for _ in range(NSLOT)]
        gsem = [next(it) for _ in range(NSLOT)]
        isem = [next(it) for _ in range(NSLOT)]
        acc = next(it)
        accc = None
        cbufs = []
        if count_mode != "none":
            accc = next(it)
            nb = 1 if count_mode == "ones" else NSLOT
            cbufs = [next(it) for _ in range(nb)]

        c = lax.axis_index("c")
        s = lax.axis_index("s")
        nj = jnp.where(c == 0, P0, P1)
        r0 = jnp.where(c == 0, s * P0, 16 * P0 + s * P1)
        z0 = s * RPS

        @pl.when(s < 15)
        def _():
            pltpu.sync_copy(zrows.at[pl.ds(0, RPS)], acc.at[pl.ds(z0, RPS)])
            if count_mode != "none":
                pltpu.sync_copy(zc.at[pl.ds(0, RPS)], accc.at[pl.ds(z0, RPS)])

        @pl.when(s == 15)
        def _():
            pltpu.sync_copy(zrows.at[pl.ds(0, RPSZ)], acc.at[pl.ds(15 * RPS, RPSZ)])
            if count_mode != "none":
                pltpu.sync_copy(zc.at[pl.ds(0, RPSZ)], accc.at[pl.ds(15 * RPS, RPSZ)])

        if count_mode == "ones":
            pltpu.sync_copy(onesrc, cbufs[0])

        def issue_idx(j, b):
            pltpu.async_copy(srcm.at[r0 + j], sidx[b], isem[b])
            pltpu.async_copy(dstm.at[r0 + j], didx[b], isem[b])

        def wait_idx(b):
            pltpu.make_async_copy(srcm.at[r0], sidx[b], isem[b]).wait()
            pltpu.make_async_copy(dstm.at[r0], didx[b], isem[b]).wait()

        def issue_gather(b):
            pltpu.async_copy(table.at[sidx[b]], bufs[b], gsem[b])
            if count_mode == "gather":
                pltpu.async_copy(cntsrc.at[sidx[b]], cbufs[b], gsem[b])

        def wait_gather(b):
            pltpu.make_async_copy(table.at[sidx[0]], bufs[b], gsem[b]).wait()
            if count_mode == "gather":
                pltpu.make_async_copy(cntsrc.at[sidx[0]], cbufs[b], gsem[b]).wait()

        def scatter(j, b):
            pltpu.sync_copy(bufs[b], acc.at[didx[b]], add=True)
            if count_mode == "ones":
                pltpu.sync_copy(cbufs[0], accc.at[didx[b]], add=True)
            elif count_mode == "gather":
                pltpu.sync_copy(cbufs[b], accc.at[didx[b]], add=True)

        plsc.subcore_barrier()

        issue_idx(0, 0)
        issue_idx(1, 1)
        wait_idx(0)
        issue_gather(0)

        def step(j2, carry):
            for b in range(NSLOT):
                j = j2 * NSLOT + b
                nb = 1 - b

                @pl.when(j < nj)
                def _():
                    @pl.when(j + 1 < nj)
                    def _():
                        wait_idx(nb)
                        issue_gather(nb)

                    wait_gather(b)
                    scatter(j, b)

                    @pl.when(j + NSLOT < nj)
                    def _():
                        issue_idx(j + NSLOT, b)
            return carry

        lax.fori_loop(0, (PMAX + NSLOT - 1) // NSLOT, step, 0)

        plsc.subcore_barrier()

        @pl.when(s < 15)
        def _():
            pltpu.sync_copy(acc.at[pl.ds(z0, RPS)], out.at[c, pl.ds(z0, RPS)])
            if count_mode != "none":
                pltpu.sync_copy(accc.at[pl.ds(z0, RPS)], outc.at[c, pl.ds(z0, RPS)])

        @pl.when(s == 15)
        def _():
            pltpu.sync_copy(
                acc.at[pl.ds(15 * RPS, RPSL)], out.at[c, pl.ds(15 * RPS, RPSL)]
            )
            if count_mode != "none":
                pltpu.sync_copy(
                    accc.at[pl.ds(15 * RPS, RPSL)], outc.at[c, pl.ds(15 * RPS, RPSL)]
                )

    return body


@functools.cache
def _sc_agg(count_mode):
    mesh = plsc.VectorSubcoreMesh(
        core_axis_name="c", subcore_axis_name="s", num_cores=2, num_subcores=16
    )
    ncb = 0 if count_mode == "none" else (1 if count_mode == "ones" else NSLOT)
    out_type = [jax.ShapeDtypeStruct((2, N, 128), jnp.float32)]
    if count_mode != "none":
        out_type.append(jax.ShapeDtypeStruct((2, N, 16), jnp.float32))
    scratch = [pltpu.VMEM((ECH,), jnp.int32) for _ in range(2 * NSLOT)]
    scratch += [pltpu.VMEM((ECH, 128), jnp.float32) for _ in range(NSLOT)]
    scratch += [pltpu.SemaphoreType.DMA for _ in range(2 * NSLOT)]
    scratch += [pltpu.VMEM_SHARED((NACC, 128), jnp.float32)]
    if count_mode != "none":
        scratch += [pltpu.VMEM_SHARED((NACC, 16), jnp.float32)]
        scratch += [pltpu.VMEM((ECH, 16), jnp.float32) for _ in range(ncb)]
    return pl.kernel(
        _make_sc_body(count_mode),
        out_type=out_type,
        mesh=mesh,
        scratch_types=scratch,
        compiler_params=pltpu.CompilerParams(use_tc_tiling_on_sc=False),
    )


def _k1_body(x_ref, wl_ref, wr_ref, p_ref, r_ref):
    xb = x_ref[...]
    p_ref[...] = jnp.dot(xb, wl_ref[...], preferred_element_type=jnp.float32)
    r_ref[...] = jnp.dot(xb, wr_ref[...], preferred_element_type=jnp.float32)


@functools.cache
def _k1():
    return pl.pallas_call(
        _k1_body,
        grid=(NBLK,),
        in_specs=[
            pl.BlockSpec((BLK, D), lambda i: (i, 0)),
            pl.BlockSpec((D, 128), lambda i: (0, 0)),
            pl.BlockSpec((D, 128), lambda i: (0, 0)),
        ],
        out_specs=[
            pl.BlockSpec((BLK, 128), lambda i: (i, 0)),
            pl.BlockSpec((BLK, 128), lambda i: (i, 0)),
        ],
        out_shape=[
            jax.ShapeDtypeStruct((N, 128), jnp.float32),
            jax.ShapeDtypeStruct((N, 128), jnp.float32),
        ],
    )


def _k2_body(parts_ref, cp_ref, r1_ref, wl_ref, wr_ref, b1_ref,
             p2_ref, r2_ref, cnt_ref):
    ps = parts_ref[0] + parts_ref[1]
    cnt = cp_ref[0, :, :1] + cp_ref[1, :, :1]
    agg = ps / jnp.maximum(cnt, 1.0)
    h = jax.nn.relu(agg + b1_ref[...] + r1_ref[...])
    p2_ref[...] = jnp.dot(h, wl_ref[...], preferred_element_type=jnp.float32)
    r2_ref[...] = jnp.dot(h, wr_ref[...], preferred_element_type=jnp.float32)
    cnt_ref[...] = cnt


@functools.cache
def _k2():
    return pl.pallas_call(
        _k2_body,
        grid=(NBLK,),
        in_specs=[
            pl.BlockSpec((2, BLK, 128), lambda i: (0, i, 0)),
            pl.BlockSpec((2, BLK, 16), lambda i: (0, i, 0)),
            pl.BlockSpec((BLK, 128), lambda i: (i, 0)),
            pl.BlockSpec((128, 128), lambda i: (0, 0)),
            pl.BlockSpec((128, 128), lambda i: (0, 0)),
            pl.BlockSpec((1, 128), lambda i: (0, 0)),
        ],
        out_specs=[
            pl.BlockSpec((BLK, 128), lambda i: (i, 0)),
            pl.BlockSpec((BLK, 128), lambda i: (i, 0)),
            pl.BlockSpec((BLK, 1), lambda i: (i, 0)),
        ],
        out_shape=[
            jax.ShapeDtypeStruct((N, 128), jnp.float32),
            jax.ShapeDtypeStruct((N, 128), jnp.float32),
            jax.ShapeDtypeStruct((N, 1), jnp.float32),
        ],
    )


def _k3a_body(parts_ref, r2_ref, cnt_ref, b2_ref, p_ref, batch_ref,
              h2_ref, s_ref, counts_ref):
    i = pl.program_id(0)
    ps = parts_ref[0] + parts_ref[1]
    agg = ps / jnp.maximum(cnt_ref[...], 1.0)
    h2 = jax.nn.relu(agg + b2_ref[...] + r2_ref[...])
    h2_ref[...] = h2
    p = p_ref[...]
    norm = jnp.sqrt(jnp.sum(p * p))
    pd = jnp.sum(h2 * p, axis=1, keepdims=True)
    s_ref[...] = jnp.tanh(pd / (norm + 1e-16))
    bt = batch_ref[0]
    gi = lax.broadcasted_iota(jnp.int32, (G, 1), 0)
    blkcnt = jnp.sum((bt == gi).astype(jnp.float32), axis=1, keepdims=True)

    @pl.when(i == 0)
    def _():
        counts_ref[...] = jnp.zeros_like(counts_ref)

    counts_ref[...] += blkcnt


@functools.cache
def _k3a():
    return pl.pallas_call(
        _k3a_body,
        grid=(NBLK,),
        in_specs=[
            pl.BlockSpec((2, BLK, 128), lambda i: (0, i, 0)),
            pl.BlockSpec((BLK, 128), lambda i: (i, 0)),
            pl.BlockSpec((BLK, 1), lambda i: (i, 0)),
            pl.BlockSpec((1, 128), lambda i: (0, 0)),
            pl.BlockSpec((1, 128), lambda i: (0, 0)),
            pl.BlockSpec((1, 1, BLK), lambda i: (i, 0, 0)),
        ],
        out_specs=[
            pl.BlockSpec((BLK, 128), lambda i: (i, 0)),
            pl.BlockSpec((BLK, 1), lambda i: (i, 0)),
            pl.BlockSpec((G, 1), lambda i: (0, 0)),
        ],
        out_shape=[
            jax.ShapeDtypeStruct((N, 128), jnp.float32),
            jax.ShapeDtypeStruct((N, 1), jnp.float32),
            jax.ShapeDtypeStruct((G, 1), jnp.float32),
        ],
    )


RBLK = 256
CCH = 512


def _krank_body(sc_ref, bc_ref, sr_ref, br_ref, counts_ref, keep_ref):
    i = pl.program_id(0)
    sc = sc_ref[...]
    bc = bc_ref[...]
    ivec = lax.broadcasted_iota(jnp.int32, (RBLK, 1), 0) + i * RBLK
    counts = counts_ref[...]
    ga = lax.broadcasted_iota(jnp.int32, (G, G), 0)
    gb = lax.broadcasted_iota(jnp.int32, (G, G), 1)
    ltri = (gb <= ga).astype(jnp.float32)
    cum = jnp.dot(ltri, counts, preferred_element_type=jnp.float32)
    startv = cum - counts
    gi = lax.broadcasted_iota(jnp.int32, (G, 1), 0)
    bmin = jnp.min(jnp.where(bc >= 0, bc, G - 1))
    bmax = jnp.max(bc)
    s_lo = jnp.sum(jnp.where(gi == bmin, startv, 0.0))
    e_hi = jnp.sum(jnp.where(gi == bmax, cum, 0.0))
    c_lo = (s_lo.astype(jnp.int32)) // CCH
    c_hi = (e_hi.astype(jnp.int32) + CCH - 1) // CCH

    def col_chunk(cc, acc):
        off = pl.multiple_of(cc * CCH, CCH)
        sj = sr_ref[:, pl.ds(off, CCH)]
        bj = br_ref[:, pl.ds(off, CCH)]
        jvec = lax.broadcasted_iota(jnp.int32, (1, CCH), 1) + cc * CCH
        beats = jnp.logical_and(
            bj == bc,
            jnp.logical_or(sj > sc, jnp.logical_and(sj == sc, jvec < ivec)),
        )
        return acc + jnp.sum(beats.astype(jnp.float32), axis=1, keepdims=True)

    acc = lax.fori_loop(c_lo, c_hi, col_chunk, jnp.zeros((RBLK, 1), jnp.float32))
    gj = lax.broadcasted_iota(jnp.int32, (1, G), 1)
    onehot = (bc == gj).astype(jnp.float32)
    kg = jnp.ceil(RATIO * counts)
    kat = jnp.dot(onehot, kg, preferred_element_type=jnp.float32)
    keep_ref[...] = (acc < kat).astype(jnp.float32)


@functools.cache
def _krank():
    return pl.pallas_call(
        _krank_body,
        grid=(NPAD // RBLK,),
        in_specs=[
            pl.BlockSpec((RBLK, 1), lambda i: (i, 0)),
            pl.BlockSpec((RBLK, 1), lambda i: (i, 0)),
            pl.BlockSpec((1, NPAD), lambda i: (0, 0)),
            pl.BlockSpec((1, NPAD), lambda i: (0, 0)),
            pl.BlockSpec((G, 1), lambda i: (0, 0)),
        ],
        out_specs=pl.BlockSpec((RBLK, 1), lambda i: (i, 0)),
        out_shape=jax.ShapeDtypeStruct((NPAD, 1), jnp.float32),
    )


def _k3c_body(h2_ref, s_ref, keep_ref, wl_ref, wr_ref, p3_ref, r_ref, k16_ref):
    keep = keep_ref[...]
    xp = h2_ref[...] * s_ref[...] * keep
    p3_ref[...] = jnp.dot(xp, wl_ref[...], preferred_element_type=jnp.float32)
    r_ref[...] = jnp.dot(xp, wr_ref[...], preferred_element_type=jnp.float32)
    k16_ref[...] = jnp.broadcast_to(keep, (BLK, 16))


@functools.cache
def _k3c():
    return pl.pallas_call(
        _k3c_body,
        grid=(NBLK,),
        in_specs=[
            pl.BlockSpec((BLK, 128), lambda i: (i, 0)),
            pl.BlockSpec((BLK, 1), lambda i: (i, 0)),
            pl.BlockSpec((BLK, 1), lambda i: (i, 0)),
            pl.BlockSpec((128, 128), lambda i: (0, 0)),
            pl.BlockSpec((128, 128), lambda i: (0, 0)),
        ],
        out_specs=[
            pl.BlockSpec((BLK, 128), lambda i: (i, 0)),
            pl.BlockSpec((BLK, 128), lambda i: (i, 0)),
            pl.BlockSpec((BLK, 16), lambda i: (i, 0)),
        ],
        out_shape=[
            jax.ShapeDtypeStruct((N, 128), jnp.float32),
            jax.ShapeDtypeStruct((N, 128), jnp.float32),
            jax.ShapeDtypeStruct((N, 16), jnp.float32),
        ],
    )


def _k4_body(parts_ref, cp_ref, r3_ref, keep_ref, batch_ref, b3_ref,
             wl1_ref, bl1_ref, wl2_ref, bl2_ref, out_ref, sums_ref, cnt2_ref):
    i = pl.program_id(0)
    ps = parts_ref[0] + parts_ref[1]
    cnt3 = cp_ref[0, :, :1] + cp_ref[1, :, :1]
    agg = ps / jnp.maximum(cnt3, 1.0)
    keep = keep_ref[...]
    h3 = jax.nn.relu(agg + b3_ref[...] + r3_ref[...]) * keep
    bt = batch_ref[0]
    gi = lax.broadcasted_iota(jnp.int32, (G, 1), 0)
    onehot = (bt == gi).astype(jnp.float32)
    bsums = jnp.dot(onehot, h3, preferred_element_type=jnp.float32)
    bcnt = jnp.dot(onehot, keep, preferred_element_type=jnp.float32)

    @pl.when(i == 0)
    def _():
        sums_ref[...] = jnp.zeros_like(sums_ref)
        cnt2_ref[...] = jnp.zeros_like(cnt2_ref)

    sums_ref[...] += bsums
    cnt2_ref[...] += bcnt

    @pl.when(i == NBLK - 1)
    def _():
        gm = sums_ref[...] / jnp.maximum(cnt2_ref[...], 1.0)
        o = jax.nn.relu(
            jnp.dot(gm, wl1_ref[...], preferred_element_type=jnp.float32)
            + bl1_ref[...]
        )
        z = jnp.dot(o, wl2_ref[...], preferred_element_type=jnp.float32) + bl2_ref[...]
        m = jnp.max(z, axis=1, keepdims=True)
        ez = jnp.exp(z - m)
        out_ref[...] = z - m - jnp.log(jnp.sum(ez, axis=1, keepdims=True))


@functools.cache
def _k4():
    return pl.pallas_call(
        _k4_body,
        grid=(NBLK,),
        in_specs=[
            pl.BlockSpec((2, BLK, 128), lambda i: (0, i, 0)),
            pl.BlockSpec((2, BLK, 16), lambda i: (0, i, 0)),
            pl.BlockSpec((BLK, 128), lambda i: (i, 0)),
            pl.BlockSpec((BLK, 1), lambda i: (i, 0)),
            pl.BlockSpec((1, 1, BLK), lambda i: (i, 0, 0)),
            pl.BlockSpec((1, 128), lambda i: (0, 0)),
            pl.BlockSpec((128, 64), lambda i: (0, 0)),
            pl.BlockSpec((1, 64), lambda i: (0, 0)),
            pl.BlockSpec((64, C), lambda i: (0, 0)),
            pl.BlockSpec((1, C), lambda i: (0, 0)),
        ],
        out_specs=pl.BlockSpec((G, C), lambda i: (0, 0)),
        out_shape=jax.ShapeDtypeStruct((G, C), jnp.float32),
        scratch_shapes=[
            pltpu.VMEM((G, 128), jnp.float32),
            pltpu.VMEM((G, 1), jnp.float32),
        ],
    )


def _agg1(table, srcm, dstm):
    zrows = jnp.zeros((RPS, 128), jnp.float32)
    zc = jnp.zeros((RPS, 16), jnp.float32)
    onesrc = jnp.ones((ECH, 16), jnp.float32)
    return _sc_agg("ones")(table, srcm, dstm, zrows, zc, onesrc)


def _agg2(table, srcm, dstm):
    zrows = jnp.zeros((RPS, 128), jnp.float32)
    return _sc_agg("none")(table, srcm, dstm, zrows)[0]


def _agg3(table, keep16, srcm, dstm):
    zrows = jnp.zeros((RPS, 128), jnp.float32)
    zc = jnp.zeros((RPS, 16), jnp.float32)
    return _sc_agg("gather")(table, srcm, dstm, zrows, zc, keep16)


def _pad_edges(ei):
    src = jnp.pad(ei[0], (0, EPAD - E)).reshape(EROWS, ECH)
    dst = jnp.pad(ei[1], (0, EPAD - E), constant_values=N).reshape(EROWS, ECH)
    return src, dst


def kernel(x, edge_index, batch, W1l, b1, W1r, W2l, b2, W2r, p_pool,
           W3l, b3, W3r, Wl1, bl1, Wl2, bl2):
    srcm, dstm = _pad_edges(edge_index)
    batch_row = batch.reshape(1, N)
    batch3 = batch.reshape(NBLK, 1, BLK)

    p1, r1 = _k1()(x, W1l, W1r)
    parts1, cnt1p = _agg1(p1, srcm, dstm)
    p2, r2, cnt1 = _k2()(parts1, cnt1p, r1, W2l, W2r, b1.reshape(1, 128))
    parts2 = _agg2(p2, srcm, dstm)
    h2, s_col, counts = _k3a()(
        parts2, r2, cnt1, b2.reshape(1, 128), p_pool.reshape(1, 128), batch3
    )

    s_row_pad = jnp.pad(s_col.reshape(1, N), ((0, 0), (0, NPAD - N)))
    s_col_pad = jnp.pad(s_col, ((0, NPAD - N), (0, 0)))
    batch_row_pad = jnp.pad(batch_row, ((0, 0), (0, NPAD - N)), constant_values=-1)
    batch_col_pad = batch_row_pad.reshape(NPAD, 1)
    keep_pad = _krank()(s_col_pad, batch_col_pad, s_row_pad, batch_row_pad, counts)
    keep = keep_pad[:N]

    p3, r3, keep16 = _k3c()(h2, s_col, keep, W3l, W3r)
    parts3, cnt3p = _agg3(p3, keep16, srcm, dstm)
    out = _k4()(
        parts3, cnt3p, r3, keep, batch3, b3.reshape(1, 128),
        Wl1, bl1.reshape(1, 64), Wl2, bl2.reshape(1, C),
    )
    return out
